# baseline (speedup 1.0000x reference)
import functools

import jax
import jax.numpy as jnp
from jax import lax
from jax.experimental import pallas as pl
from jax.experimental.pallas import tpu as pltpu
from jax.experimental.pallas import tpu_sc as plsc

_CHUNK = 80
_NBUF = 3
_BLK = 1024
_D_OUT = 256


def _sc_gather(x_t, emb_var, emb_vc, emb_func):
    info = plsc.get_sparse_core_info()
    nc, ns = info.num_cores, info.num_subcores
    nw = nc * ns
    n_tok = x_t.shape[1]
    per_w = n_tok // nw
    n_chunks = per_w // _CHUNK
    n_groups = _CHUNK // 16

    mesh = plsc.VectorSubcoreMesh(core_axis_name="c", subcore_axis_name="s")

    @functools.partial(
        pl.kernel,
        mesh=mesh,
        compiler_params=pltpu.CompilerParams(
            use_tc_tiling_on_sc=False, needs_layout_passes=False),
        out_type=[
            jax.ShapeDtypeStruct((n_tok, 128), jnp.float32),
            jax.ShapeDtypeStruct((n_tok, 128), jnp.float32),
        ],
        scratch_types=[
            pltpu.VMEM((_NBUF, 9, _CHUNK), jnp.float32),
            pltpu.VMEM((_NBUF, _CHUNK), jnp.int32),
            pltpu.VMEM((_NBUF, _CHUNK), jnp.int32),
            pltpu.VMEM((_NBUF, 6, _CHUNK), jnp.int32),
            pltpu.VMEM((_NBUF, _CHUNK, 128), jnp.float32),
            pltpu.VMEM((_NBUF, _CHUNK, 32), jnp.float32),
            pltpu.VMEM((_NBUF, 6, _CHUNK, 32), jnp.float32),
            pltpu.VMEM((_NBUF, _CHUNK, 128), jnp.float32),
            pltpu.SemaphoreType.DMA((_NBUF,)),
            pltpu.SemaphoreType.DMA((_NBUF,)),
        ],
    )
    def k(xt_h, table_h, vc_tab_h, func_tab_h, hvar_h, hsmall_h,
          xch, vidx, vcidx, fidx, vrows, vcrows, frows, small, ssem, wsem):
        wid = lax.axis_index("s") * nc + lax.axis_index("c")
        iota16 = lax.iota(jnp.int32, 16)
        zero16 = jnp.zeros((16,), jnp.float32)
        col64 = jnp.full((16,), 64, jnp.int32)

        def zero_body(t, carry):
            for sbuf in range(_NBUF):
                for h in range(4):
                    small[sbuf, t, pl.ds(64 + h * 16, 16)] = zero16
            return carry

        lax.fori_loop(0, _CHUNK, zero_body, 0)

        def stage_and_fire(g, s):
            base = wid * per_w + g * _CHUNK
            pltpu.sync_copy(xt_h.at[:, pl.ds(base, _CHUNK)], xch.at[s])

            def idx_body(gg, carry2):
                sl = pl.ds(gg * 16, 16)
                vidx[s, sl] = xch[s, 0, sl].astype(jnp.int32)
                vcidx[s, sl] = xch[s, 1, sl].astype(jnp.int32)
                for j in range(6):
                    fidx[s, j, sl] = xch[s, 2 + j, sl].astype(jnp.int32)
                return carry2

            lax.fori_loop(0, n_groups, idx_body, 0)
            pltpu.async_copy(table_h.at[vidx.at[s]], vrows.at[s], ssem.at[s])
            pltpu.async_copy(vc_tab_h.at[vcidx.at[s]], vcrows.at[s], ssem.at[s])
            for j in range(6):
                pltpu.async_copy(func_tab_h.at[fidx.at[s, j]],
                                 frows.at[s, j], ssem.at[s])

        def wait_streams(s):
            pltpu.make_async_copy(table_h.at[vidx.at[s]], vrows.at[s],
                                  ssem.at[s]).wait()
            pltpu.make_async_copy(vc_tab_h.at[vcidx.at[s]], vcrows.at[s],
                                  ssem.at[s]).wait()
            for j in range(6):
                pltpu.make_async_copy(func_tab_h.at[fidx.at[s, j]],
                                      frows.at[s, j], ssem.at[s]).wait()

        def pool(s):
            def pool_body(t, carry2):
                for h in range(2):
                    sl = pl.ds(h * 16, 16)
                    small[s, t, sl] = vcrows[s, t, sl]
                    acc = frows[s, 0, t, sl]
                    for j in range(1, 6):
                        acc = acc + frows[s, j, t, sl]
                    small[s, t, pl.ds(32 + h * 16, 16)] = acc * (1.0 / 6.0)
                return carry2

            lax.fori_loop(0, _CHUNK, pool_body, 0)

            def vaf_body(gg, carry2):
                t0 = gg * 16
                vafv = xch[s, 8, pl.ds(t0, 16)]
                plsc.store_scatter(small.at[s], [iota16 + t0, col64], vafv)
                return carry2

            lax.fori_loop(0, n_groups, vaf_body, 0)

        def write_out(g, s):
            base = wid * per_w + g * _CHUNK
            pltpu.async_copy(vrows.at[s], hvar_h.at[pl.ds(base, _CHUNK)],
                             wsem.at[s])
            pltpu.async_copy(small.at[s], hsmall_h.at[pl.ds(base, _CHUNK)],
                             wsem.at[s])

        def drain_writes(s):
            base = wid * per_w
            pltpu.make_async_copy(vrows.at[s], hvar_h.at[pl.ds(base, _CHUNK)],
                                  wsem.at[s]).wait()
            pltpu.make_async_copy(small.at[s], hsmall_h.at[pl.ds(base, _CHUNK)],
                                  wsem.at[s]).wait()

        def slot(g, s, drain_guarded, fire_next):
            wait_streams(s)
            pool(s)
            prev = (s + _NBUF - 1) % _NBUF
            if drain_guarded:
                @pl.when(g >= 1)
                def _():
                    drain_writes(prev)
            else:
                drain_writes(prev)
            write_out(g, s)
            if fire_next:
                stage_and_fire(g + 2, (s + 2) % _NBUF)

        stage_and_fire(0, 0)
        stage_and_fire(1, 1)

        n_body = (n_chunks - 2) // _NBUF

        def body(gg, carry):
            g0 = gg * _NBUF
            for s in range(_NBUF):
                slot(g0 + s, s, s == 0, True)
            return carry

        lax.fori_loop(0, n_body, body, 0)
        for g in range(n_body * _NBUF, n_chunks):
            slot(g, g % _NBUF, False, g + 2 < n_chunks)
        drain_writes((n_chunks - 1) % _NBUF)

    return k(x_t, emb_var, emb_vc, emb_func)


def _tc_body(hv_ref, hsm_ref, w_ref, wsm_ref, b_ref, out_ref):
    wv = w_ref[0:128, :]
    acc = jnp.dot(hv_ref[...], wv, preferred_element_type=jnp.float32)
    acc = acc + jnp.dot(hsm_ref[...], wsm_ref[...],
                        preferred_element_type=jnp.float32)
    acc = acc + b_ref[...]
    out_ref[...] = jnp.where(acc > 0.0, acc,
                             jnp.exp(jnp.minimum(acc, 0.0)) - 1.0)


def _tc_project(hvar, hsmall, w, wsm_pad, b2):
    n_tok = hvar.shape[0]
    grid = (n_tok // _BLK,)
    return pl.pallas_call(
        _tc_body,
        grid=grid,
        in_specs=[
            pl.BlockSpec((_BLK, 128), lambda i: (i, 0)),
            pl.BlockSpec((_BLK, 128), lambda i: (i, 0)),
            pl.BlockSpec((193, _D_OUT), lambda i: (0, 0)),
            pl.BlockSpec((128, _D_OUT), lambda i: (0, 0)),
            pl.BlockSpec((1, _D_OUT), lambda i: (0, 0)),
        ],
        out_specs=pl.BlockSpec((_BLK, _D_OUT), lambda i: (i, 0)),
        out_shape=jax.ShapeDtypeStruct((n_tok, _D_OUT), jnp.float32),
        compiler_params=pltpu.CompilerParams(
            dimension_semantics=("parallel",)),
    )(hvar, hsmall, w, wsm_pad, b2)


def kernel(x_omic, emb_var, emb_vc, emb_func, W, b):
    bsz, seq, _ = x_omic.shape
    n_tok = bsz * seq
    x_t = x_omic.transpose(2, 1, 0).reshape(9, n_tok)
    wsm_pad = jnp.pad(W[128:193], ((0, 63), (0, 0)))
    hvar, hsmall = _sc_gather(x_t, emb_var, emb_vc, emb_func)
    out2d = _tc_project(hvar, hsmall, W, wsm_pad, b.reshape(1, -1))
    return out2d.reshape(seq, bsz, _D_OUT).transpose(1, 0, 2)


# speedup vs baseline: 1.3648x; 1.3648x over previous
import functools

import jax
import jax.numpy as jnp
from jax import lax
from jax.experimental import pallas as pl
from jax.experimental.pallas import tpu as pltpu
from jax.experimental.pallas import tpu_sc as plsc

_CHUNK = 80
_NBUF = 3
_SMW = 130
_BLK = 1024
_D_OUT = 256


def _sc_gather(x_t, emb_var, ef_pad, evc_pad):
    info = plsc.get_sparse_core_info()
    nc, ns = info.num_cores, info.num_subcores
    nw = nc * ns
    n_tok = x_t.shape[1]
    per_w = n_tok // nw
    n_chunks = per_w // _CHUNK
    n_groups = _CHUNK // 16

    mesh = plsc.VectorSubcoreMesh(core_axis_name="c", subcore_axis_name="s")

    @functools.partial(
        pl.kernel,
        mesh=mesh,
        compiler_params=pltpu.CompilerParams(
            use_tc_tiling_on_sc=False, needs_layout_passes=False),
        out_type=[
            jax.ShapeDtypeStruct((n_tok, 128), jnp.float32),
            jax.ShapeDtypeStruct((n_tok, 128), jnp.float32),
        ],
        scratch_types=[
            pltpu.VMEM((_NBUF, 9, _CHUNK), jnp.float32),
            pltpu.VMEM((_NBUF, _CHUNK), jnp.int32),
            pltpu.VMEM((_NBUF, _CHUNK, 128), jnp.float32),
            pltpu.VMEM((_NBUF, _CHUNK, _SMW), jnp.float32),
            pltpu.VMEM(ef_pad.shape, jnp.float32),
            pltpu.VMEM(evc_pad.shape, jnp.float32),
            pltpu.SemaphoreType.DMA((_NBUF,)),
            pltpu.SemaphoreType.DMA((_NBUF,)),
        ],
    )
    def k(xt_h, table_h, ef_h, evc_h, hvar_h, hsmall_h,
          xch, vidx, vrows, small, ef_v, evc_v, ssem, wsem):
        wid = lax.axis_index("s") * nc + lax.axis_index("c")
        iota16 = lax.iota(jnp.int32, 16)
        zero16 = jnp.zeros((16,), jnp.float32)
        col64 = jnp.full((16,), 64, jnp.int32)

        pltpu.sync_copy(ef_h, ef_v)
        pltpu.sync_copy(evc_h, evc_v)

        def zero_body(t, carry):
            for sbuf in range(_NBUF):
                for h in range(4):
                    small[sbuf, t, pl.ds(64 + h * 16, 16)] = zero16
            return carry

        lax.fori_loop(0, _CHUNK, zero_body, 0)

        def stage_and_fire(g, s):
            base = wid * per_w + g * _CHUNK
            pltpu.sync_copy(xt_h.at[:, pl.ds(base, _CHUNK)], xch.at[s])

            def idx_body(gg, carry2):
                sl = pl.ds(gg * 16, 16)
                vidx[s, sl] = xch[s, 0, sl].astype(jnp.int32)
                return carry2

            lax.fori_loop(0, n_groups, idx_body, 0)
            pltpu.async_copy(table_h.at[vidx.at[s]], vrows.at[s], ssem.at[s])

        def wait_streams(s):
            pltpu.make_async_copy(table_h.at[vidx.at[s]], vrows.at[s],
                                  ssem.at[s]).wait()

        def pool(s):
            def group_body(gg, carry2):
                t0 = gg * 16
                sl = pl.ds(t0, 16)
                toks = iota16 + t0
                vcid = xch[s, 1, sl].astype(jnp.int32)
                fids = [xch[s, 2 + j, sl].astype(jnp.int32) for j in range(6)]
                vafv = xch[s, 8, sl]
                plsc.store_scatter(small.at[s], [toks, col64], vafv)
                for c in range(32):
                    colv = jnp.full((16,), c, jnp.int32)
                    vcv = plsc.load_gather(evc_v, [vcid, colv])
                    acc = plsc.load_gather(ef_v, [fids[0], colv])
                    for j in range(1, 6):
                        acc = acc + plsc.load_gather(ef_v, [fids[j], colv])
                    plsc.store_scatter(small.at[s], [toks, colv], vcv)
                    plsc.store_scatter(
                        small.at[s], [toks, jnp.full((16,), c + 32, jnp.int32)],
                        acc * (1.0 / 6.0))
                return carry2

            lax.fori_loop(0, n_groups, group_body, 0)

        def write_out(g, s):
            base = wid * per_w + g * _CHUNK
            pltpu.async_copy(vrows.at[s], hvar_h.at[pl.ds(base, _CHUNK)],
                             wsem.at[s])
            pltpu.async_copy(small.at[s, :, pl.ds(0, 128)],
                             hsmall_h.at[pl.ds(base, _CHUNK)], wsem.at[s])

        def drain_writes(s):
            base = wid * per_w
            pltpu.make_async_copy(vrows.at[s], hvar_h.at[pl.ds(base, _CHUNK)],
                                  wsem.at[s]).wait()
            pltpu.make_async_copy(small.at[s, :, pl.ds(0, 128)],
                                  hsmall_h.at[pl.ds(base, _CHUNK)],
                                  wsem.at[s]).wait()

        def slot(g, s, drain_guarded, fire_next):
            pool(s)
            wait_streams(s)
            prev = (s + _NBUF - 1) % _NBUF
            if drain_guarded:
                @pl.when(g >= 1)
                def _():
                    drain_writes(prev)
            else:
                drain_writes(prev)
            write_out(g, s)
            if fire_next:
                stage_and_fire(g + 2, (s + 2) % _NBUF)

        stage_and_fire(0, 0)
        stage_and_fire(1, 1)

        n_body = (n_chunks - 2) // _NBUF

        def body(gg, carry):
            g0 = gg * _NBUF
            for s in range(_NBUF):
                slot(g0 + s, s, s == 0, True)
            return carry

        lax.fori_loop(0, n_body, body, 0)
        for g in range(n_body * _NBUF, n_chunks):
            slot(g, g % _NBUF, False, g + 2 < n_chunks)
        drain_writes((n_chunks - 1) % _NBUF)

    return k(x_t, emb_var, ef_pad, evc_pad)


def _tc_body(hv_ref, hsm_ref, w_ref, wsm_ref, b_ref, out_ref):
    wv = w_ref[0:128, :]
    acc = jnp.dot(hv_ref[...], wv, preferred_element_type=jnp.float32)
    acc = acc + jnp.dot(hsm_ref[...], wsm_ref[...],
                        preferred_element_type=jnp.float32)
    acc = acc + b_ref[...]
    out_ref[...] = jnp.where(acc > 0.0, acc,
                             jnp.exp(jnp.minimum(acc, 0.0)) - 1.0)


def _tc_project(hvar, hsmall, w, wsm_pad, b2):
    n_tok = hvar.shape[0]
    grid = (n_tok // _BLK,)
    return pl.pallas_call(
        _tc_body,
        grid=grid,
        in_specs=[
            pl.BlockSpec((_BLK, 128), lambda i: (i, 0)),
            pl.BlockSpec((_BLK, 128), lambda i: (i, 0)),
            pl.BlockSpec((193, _D_OUT), lambda i: (0, 0)),
            pl.BlockSpec((128, _D_OUT), lambda i: (0, 0)),
            pl.BlockSpec((1, _D_OUT), lambda i: (0, 0)),
        ],
        out_specs=pl.BlockSpec((_BLK, _D_OUT), lambda i: (i, 0)),
        out_shape=jax.ShapeDtypeStruct((n_tok, _D_OUT), jnp.float32),
        compiler_params=pltpu.CompilerParams(
            dimension_semantics=("parallel",)),
    )(hvar, hsmall, w, wsm_pad, b2)


def kernel(x_omic, emb_var, emb_vc, emb_func, W, b):
    bsz, seq, _ = x_omic.shape
    n_tok = bsz * seq
    x_t = x_omic.transpose(2, 1, 0).reshape(9, n_tok)
    ef_pad = jnp.pad(emb_func, ((0, 7), (0, 1)))
    evc_pad = jnp.pad(emb_vc, ((0, 5), (0, 1)))
    wsm_pad = jnp.pad(W[128:193], ((0, 63), (0, 0)))
    hvar, hsmall = _sc_gather(x_t, emb_var, ef_pad, evc_pad)
    out2d = _tc_project(hvar, hsmall, W, wsm_pad, b.reshape(1, -1))
    return out2d.reshape(seq, bsz, _D_OUT).transpose(1, 0, 2)


# speedup vs baseline: 1.4776x; 1.0827x over previous
import functools

import jax
import jax.numpy as jnp
from jax import lax
from jax.experimental import pallas as pl
from jax.experimental.pallas import tpu as pltpu
from jax.experimental.pallas import tpu_sc as plsc

_CHUNK = 80
_NBUF = 3
_SMW = 130
_BLK = 1024
_D_OUT = 256


def _sc_gather(x_t, emb_var, ef_pad, evc_pad):
    info = plsc.get_sparse_core_info()
    nc, ns = info.num_cores, info.num_subcores
    nw = nc * ns
    n_tok = x_t.shape[1]
    per_w = n_tok // nw
    n_chunks = per_w // _CHUNK
    n_groups = _CHUNK // 16

    mesh = plsc.VectorSubcoreMesh(core_axis_name="c", subcore_axis_name="s")

    @functools.partial(
        pl.kernel,
        mesh=mesh,
        compiler_params=pltpu.CompilerParams(
            use_tc_tiling_on_sc=False, needs_layout_passes=False),
        out_type=[
            jax.ShapeDtypeStruct((n_tok, 128), jnp.float32),
            jax.ShapeDtypeStruct((n_tok, 128), jnp.float32),
        ],
        scratch_types=[
            pltpu.VMEM((_NBUF, 9, _CHUNK), jnp.float32),
            pltpu.VMEM((_NBUF, _CHUNK), jnp.int32),
            pltpu.VMEM((_NBUF, _CHUNK, 128), jnp.float32),
            pltpu.VMEM((_NBUF, _CHUNK, _SMW), jnp.float32),
            pltpu.VMEM(ef_pad.shape, jnp.float32),
            pltpu.VMEM(evc_pad.shape, jnp.float32),
            pltpu.SemaphoreType.DMA((_NBUF,)),
            pltpu.SemaphoreType.DMA((_NBUF,)),
            pltpu.SemaphoreType.DMA((_NBUF,)),
        ],
    )
    def k(xt_h, table_h, ef_h, evc_h, hvar_h, hsmall_h,
          xch, vidx, vrows, small, ef_v, evc_v, ssem, wsem, xsem):
        wid = lax.axis_index("s") * nc + lax.axis_index("c")
        iota16 = lax.iota(jnp.int32, 16)
        zero16 = jnp.zeros((16,), jnp.float32)
        col64 = jnp.full((16,), 64, jnp.int32)

        pltpu.sync_copy(ef_h, ef_v)
        pltpu.sync_copy(evc_h, evc_v)

        def zero_body(t, carry):
            for sbuf in range(_NBUF):
                for h in range(4):
                    small[sbuf, t, pl.ds(64 + h * 16, 16)] = zero16
            return carry

        lax.fori_loop(0, _CHUNK, zero_body, 0)

        def stage_x(g, s):
            base = wid * per_w + g * _CHUNK
            pltpu.async_copy(xt_h.at[:, pl.ds(base, _CHUNK)], xch.at[s],
                             xsem.at[s])

        def wait_x(s):
            base = wid * per_w
            pltpu.make_async_copy(xt_h.at[:, pl.ds(base, _CHUNK)], xch.at[s],
                                  xsem.at[s]).wait()

        def fire_var(s):
            def idx_body(gg, carry2):
                sl = pl.ds(gg * 16, 16)
                vidx[s, sl] = xch[s, 0, sl].astype(jnp.int32)
                return carry2

            lax.fori_loop(0, n_groups, idx_body, 0)
            pltpu.async_copy(table_h.at[vidx.at[s]], vrows.at[s], ssem.at[s])

        def wait_var(s):
            pltpu.make_async_copy(table_h.at[vidx.at[s]], vrows.at[s],
                                  ssem.at[s]).wait()

        def pool(s):
            def group_body(gg, carry2):
                t0 = gg * 16
                sl = pl.ds(t0, 16)
                toks = iota16 + t0
                vcid = xch[s, 1, sl].astype(jnp.int32)
                fids = [xch[s, 2 + j, sl].astype(jnp.int32) for j in range(6)]
                vafv = xch[s, 8, sl]
                plsc.store_scatter(small.at[s], [toks, col64], vafv)
                for c in range(32):
                    colv = jnp.full((16,), c, jnp.int32)
                    vcv = plsc.load_gather(evc_v, [vcid, colv])
                    acc = plsc.load_gather(ef_v, [fids[0], colv])
                    for j in range(1, 6):
                        acc = acc + plsc.load_gather(ef_v, [fids[j], colv])
                    plsc.store_scatter(small.at[s], [toks, colv], vcv)
                    plsc.store_scatter(
                        small.at[s], [toks, jnp.full((16,), c + 32, jnp.int32)],
                        acc * (1.0 / 6.0))
                return carry2

            lax.fori_loop(0, n_groups, group_body, 0)

        def write_out(g, s):
            base = wid * per_w + g * _CHUNK
            pltpu.async_copy(vrows.at[s], hvar_h.at[pl.ds(base, _CHUNK)],
                             wsem.at[s])
            pltpu.async_copy(small.at[s, :, pl.ds(0, 128)],
                             hsmall_h.at[pl.ds(base, _CHUNK)], wsem.at[s])

        def drain_writes(s):
            base = wid * per_w
            pltpu.make_async_copy(vrows.at[s], hvar_h.at[pl.ds(base, _CHUNK)],
                                  wsem.at[s]).wait()
            pltpu.make_async_copy(small.at[s, :, pl.ds(0, 128)],
                                  hsmall_h.at[pl.ds(base, _CHUNK)],
                                  wsem.at[s]).wait()

        def slot(g, s, drain_guarded, has_next, has_next2):
            if has_next:
                s1 = (s + 1) % _NBUF
                wait_x(s1)
                fire_var(s1)
            if has_next2:
                stage_x(g + 2, (s + 2) % _NBUF)
            pool(s)
            wait_var(s)
            prev = (s + _NBUF - 1) % _NBUF
            if drain_guarded:
                @pl.when(g >= 1)
                def _():
                    drain_writes(prev)
            else:
                drain_writes(prev)
            write_out(g, s)

        stage_x(0, 0)
        stage_x(1, 1)
        wait_x(0)
        fire_var(0)

        n_body = (n_chunks - 2) // _NBUF

        def body(gg, carry):
            g0 = gg * _NBUF
            for s in range(_NBUF):
                slot(g0 + s, s, s == 0, True, True)
            return carry

        lax.fori_loop(0, n_body, body, 0)
        for g in range(n_body * _NBUF, n_chunks):
            slot(g, g % _NBUF, False, g + 1 < n_chunks, g + 2 < n_chunks)
        drain_writes((n_chunks - 1) % _NBUF)

    return k(x_t, emb_var, ef_pad, evc_pad)


def _tc_body(hv_ref, hsm_ref, w_ref, wsm_ref, b_ref, out_ref):
    wv = w_ref[0:128, :]
    acc = jnp.dot(hv_ref[...], wv, preferred_element_type=jnp.float32)
    acc = acc + jnp.dot(hsm_ref[...], wsm_ref[...],
                        preferred_element_type=jnp.float32)
    acc = acc + b_ref[...]
    out_ref[...] = jnp.where(acc > 0.0, acc,
                             jnp.exp(jnp.minimum(acc, 0.0)) - 1.0)


def _tc_project(hvar, hsmall, w, wsm_pad, b2):
    n_tok = hvar.shape[0]
    grid = (n_tok // _BLK,)
    return pl.pallas_call(
        _tc_body,
        grid=grid,
        in_specs=[
            pl.BlockSpec((_BLK, 128), lambda i: (i, 0)),
            pl.BlockSpec((_BLK, 128), lambda i: (i, 0)),
            pl.BlockSpec((193, _D_OUT), lambda i: (0, 0)),
            pl.BlockSpec((128, _D_OUT), lambda i: (0, 0)),
            pl.BlockSpec((1, _D_OUT), lambda i: (0, 0)),
        ],
        out_specs=pl.BlockSpec((_BLK, _D_OUT), lambda i: (i, 0)),
        out_shape=jax.ShapeDtypeStruct((n_tok, _D_OUT), jnp.float32),
        compiler_params=pltpu.CompilerParams(
            dimension_semantics=("parallel",)),
    )(hvar, hsmall, w, wsm_pad, b2)


def kernel(x_omic, emb_var, emb_vc, emb_func, W, b):
    bsz, seq, _ = x_omic.shape
    n_tok = bsz * seq
    x_t = x_omic.transpose(2, 1, 0).reshape(9, n_tok)
    ef_pad = jnp.pad(emb_func, ((0, 7), (0, 1)))
    evc_pad = jnp.pad(emb_vc, ((0, 5), (0, 1)))
    wsm_pad = jnp.pad(W[128:193], ((0, 63), (0, 0)))
    hvar, hsmall = _sc_gather(x_t, emb_var, ef_pad, evc_pad)
    out2d = _tc_project(hvar, hsmall, W, wsm_pad, b.reshape(1, -1))
    return out2d.reshape(seq, bsz, _D_OUT).transpose(1, 0, 2)
